# baseline (device time: 173963 ns/iter reference)
import functools

import jax
import jax.numpy as jnp
from jax import lax
from jax.experimental import pallas as pl
from jax.experimental.pallas import tpu as pltpu

N_DEV = 8
SQ = 256
SKV = 2048
HQ = 8
DH = 128
DM = 1024
SCALE = 0.08838834764831843


def _attn_partial(c, xc, wq_ref, k_ref, v_ref, wo_ref, qbuf):
    qbuf[...] = lax.dot_general(
        xc, wq_ref[...], (((1,), (0,)), ((), ())),
        preferred_element_type=jnp.float32,
    ).astype(jnp.bfloat16)

    rows = lax.broadcasted_iota(jnp.int32, (SQ, SKV), 0) + c * SQ
    cols = lax.broadcasted_iota(jnp.int32, (SQ, SKV), 1)
    keep = (cols // 64) <= (rows // 64)

    def h_body(h, acc):
        qh = qbuf[:, pl.ds(h * DH, DH)]
        s = lax.dot_general(
            qh, k_ref[h], (((1,), (1,)), ((), ())),
            preferred_element_type=jnp.float32,
        ) * SCALE
        s = jnp.where(keep, s, -1e9)
        m = jnp.max(s, axis=1, keepdims=True)
        w = jnp.exp(s - m)
        w = (w / jnp.sum(w, axis=1, keepdims=True)).astype(jnp.bfloat16)
        ctx = lax.dot_general(
            w, v_ref[h], (((1,), (0,)), ((), ())),
            preferred_element_type=jnp.float32,
        ).astype(jnp.bfloat16)
        return acc + lax.dot_general(
            ctx, wo_ref[pl.ds(h * DH, DH), :], (((1,), (0,)), ((), ())),
            preferred_element_type=jnp.float32,
        )

    return lax.fori_loop(0, HQ, h_body, jnp.zeros((SQ, DM), jnp.float32))


def _body(x_ref, wq_ref, k_ref, v_ref, wo_ref, out_ref,
          xR, xL, rsR, rsL, qbuf,
          agR_s, agR_r, agL_s, agL_r, rsR_s, rsR_r, rsL_s, rsL_r):
    i = lax.axis_index("i")
    right = lax.rem(i + 1, N_DEV)
    left = lax.rem(i - 1 + N_DEV, N_DEV)

    def attn(c, xc):
        return _attn_partial(c, xc, wq_ref, k_ref, v_ref, wo_ref, qbuf)

    def rdma(buf, src_slot, dst_slot, send_sem, recv_sem, idx, dev):
        return pltpu.make_async_remote_copy(
            src_ref=buf.at[src_slot],
            dst_ref=buf.at[dst_slot],
            send_sem=send_sem.at[idx],
            recv_sem=recv_sem.at[idx],
            device_id=(dev,),
            device_id_type=pl.DeviceIdType.MESH,
        )

    def agR(t):
        return rdma(xR, t, t + 1, agR_s, agR_r, t, right)

    def agL(t):
        return rdma(xL, t, t + 1, agL_s, agL_r, t, left)

    def rsRd(u):
        return rdma(rsR, 3 if u == 0 else u - 1, u, rsR_s, rsR_r, u, right)

    def rsLd(v):
        return rdma(rsL, 4 if v == 0 else v - 1, v, rsL_s, rsL_r, v, left)

    xR[0] = x_ref[...]
    xL[0] = x_ref[...]
    agR(0).start()
    agL(0).start()
    partial_own = attn(i, x_ref[...])

    pR = []
    pL = []
    for t in range(3):
        agR(t).wait_recv()
        agR(t + 1).start()
        agL(t).wait_recv()
        if t < 2:
            agL(t + 1).start()
        pR.append(attn(lax.rem(i - t - 1 + N_DEV, N_DEV), xR[t + 1]))
        pL.append(attn(lax.rem(i + t + 1, N_DEV), xL[t + 1]))

    rsR[3] = pL[2].astype(jnp.bfloat16)
    rsRd(0).start()

    agR(3).wait_recv()
    pR.append(attn(lax.rem(i - 4 + N_DEV, N_DEV), xR[4]))
    rsL[4] = pR[3].astype(jnp.bfloat16)
    rsLd(0).start()

    rsRd(0).wait_recv()
    rsR[0] = (rsR[0] + pL[1]).astype(jnp.bfloat16)
    rsRd(1).start()
    rsLd(0).wait_recv()
    rsL[0] = (rsL[0] + pR[2]).astype(jnp.bfloat16)
    rsLd(1).start()
    rsRd(1).wait_recv()
    rsR[1] = (rsR[1] + pL[0]).astype(jnp.bfloat16)
    rsRd(2).start()
    rsLd(1).wait_recv()
    rsL[1] = (rsL[1] + pR[1]).astype(jnp.bfloat16)
    rsLd(2).start()
    rsLd(2).wait_recv()
    rsL[2] = (rsL[2] + pR[0]).astype(jnp.bfloat16)
    rsLd(3).start()

    rsRd(2).wait_recv()
    rsLd(3).wait_recv()
    out_ref[...] = rsR[2] + rsL[3] + partial_own

    for t in range(4):
        agR(t).wait_send()
    for t in range(3):
        agL(t).wait_send()
        rsRd(t).wait_send()
    for v in range(4):
        rsLd(v).wait_send()


def kernel(x, Wq, K_ext, V_ext, Wo):
    i = lax.axis_index("i")
    k_sl = lax.dynamic_slice(K_ext, (0, 0, i * HQ, 0), (1, SKV, HQ, DH))[0]
    v_sl = lax.dynamic_slice(V_ext, (0, 0, i * HQ, 0), (1, SKV, HQ, DH))[0]
    k_hm = jnp.transpose(k_sl, (1, 0, 2)).astype(jnp.bfloat16)
    v_hm = jnp.transpose(v_sl, (1, 0, 2)).astype(jnp.bfloat16)

    out = pl.pallas_call(
        _body,
        out_shape=jax.ShapeDtypeStruct((SQ, DM), jnp.float32),
        in_specs=[pl.BlockSpec(memory_space=pltpu.VMEM)] * 5,
        out_specs=pl.BlockSpec(memory_space=pltpu.VMEM),
        scratch_shapes=[
            pltpu.VMEM((5, SQ, DM), jnp.bfloat16),
            pltpu.VMEM((4, SQ, DM), jnp.bfloat16),
            pltpu.VMEM((4, SQ, DM), jnp.bfloat16),
            pltpu.VMEM((5, SQ, DM), jnp.bfloat16),
            pltpu.VMEM((SQ, HQ * DH), jnp.bfloat16),
            pltpu.SemaphoreType.DMA((4,)),
            pltpu.SemaphoreType.DMA((4,)),
            pltpu.SemaphoreType.DMA((3,)),
            pltpu.SemaphoreType.DMA((3,)),
            pltpu.SemaphoreType.DMA((3,)),
            pltpu.SemaphoreType.DMA((3,)),
            pltpu.SemaphoreType.DMA((4,)),
            pltpu.SemaphoreType.DMA((4,)),
        ],
    )(x[0].astype(jnp.bfloat16), Wq.astype(jnp.bfloat16), k_hm, v_hm,
      Wo.astype(jnp.bfloat16))
    return out.reshape(1, SQ, DM)
